# baseline (speedup 1.0000x reference)
import functools

import jax
import jax.numpy as jnp
from jax import lax
from jax.experimental import pallas as pl
from jax.experimental.pallas import tpu as pltpu
from jax.experimental.pallas import tpu_sc as plsc

_BS, _V, _NB = 2, 10000, 16
_SK, _K = 256, 128
_VB = 2048


def _sc_gather_dirs(vert_planar, idx_t):
    mesh = plsc.VectorSubcoreMesh(core_axis_name="c", subcore_axis_name="s")

    @functools.partial(
        pl.kernel,
        out_type=jax.ShapeDtypeStruct((_BS, _NB, 3, _V), jnp.float32),
        mesh=mesh,
        scratch_types=[
            [pltpu.VMEM((_V,), jnp.float32) for _ in range(3)],
            pltpu.VMEM((_V,), jnp.int32),
            pltpu.VMEM((3, _V), jnp.float32),
        ],
        compiler_params=pltpu.CompilerParams(needs_layout_passes=False),
    )
    def k(vert_hbm, idx_hbm, out_hbm, tabs, idxs, outs):
        cid = lax.axis_index("c")
        sid = lax.axis_index("s")
        w = sid * 2 + cid
        b = w // _NB
        n = w % _NB
        for c in range(3):
            pltpu.sync_copy(vert_hbm.at[pl.ds((b * 3 + c) * _V, _V)], tabs[c])
        pltpu.sync_copy(idx_hbm.at[pl.ds((b * _NB + n) * _V, _V)], idxs)

        def body(i, carry):
            for u in range(5):
                s = (i * 5 + u) * 16
                iv = idxs[pl.ds(s, 16)]
                for c in range(3):
                    g = plsc.load_gather(tabs[c], [iv])
                    outs[c, pl.ds(s, 16)] = g - tabs[c][pl.ds(s, 16)]
            return carry

        lax.fori_loop(0, _V // 80, body, 0)
        pltpu.sync_copy(outs, out_hbm.at[b, n])

    return k(vert_planar, idx_t)


def _tc_dense(dirs, w_t, dw_t):
    nblk = (_V + _VB - 1) // _VB

    def body(dirs_ref, w_ref, dw_ref, out_ref):
        wv = w_ref[...]
        wn = wv / jnp.maximum(
            jnp.sqrt(jnp.sum(wv * wv, axis=1, keepdims=True)), 1e-12
        )
        acc = None
        dist = None
        for n in range(_NB):
            a = dirs_ref[0, n]
            sq = a[0:1, :] ** 2 + a[1:2, :] ** 2 + a[2:3, :] ** 2
            nrm = jnp.sqrt(sq)
            inv = 1.0 / jnp.maximum(nrm, 1e-12)
            th = jnp.dot(
                wn.astype(jnp.bfloat16),
                (a * inv).astype(jnp.bfloat16),
                preferred_element_type=jnp.float32,
            ).astype(jnp.bfloat16)
            acc = th if acc is None else jnp.maximum(acc, th)
            dist = nrm if dist is None else jnp.maximum(dist, nrm)
        acc = jnp.maximum(acc.astype(jnp.float32), 0.0)
        dv = jnp.maximum(dw_ref[...] * dist, 0.0)
        f = acc + dv
        out_ref[0] = (f[:_K, :] + f[_K:, :]).T

    return pl.pallas_call(
        body,
        grid=(_BS, nblk),
        in_specs=[
            pl.BlockSpec((1, _NB, 3, _VB), lambda b, i: (b, 0, 0, i)),
            pl.BlockSpec((_SK, 3), lambda b, i: (0, 0)),
            pl.BlockSpec((_SK, 1), lambda b, i: (0, 0)),
        ],
        out_specs=pl.BlockSpec((1, _VB, _K), lambda b, i: (b, i, 0)),
        out_shape=jax.ShapeDtypeStruct((_BS, _V, _K), jnp.float32),
    )(dirs, w_t, dw_t)


def kernel(neighbor_index, vertices, directions, distance):
    vert_planar = vertices.transpose(0, 2, 1).reshape(-1)
    idx_t = neighbor_index.transpose(0, 2, 1).astype(jnp.int32).reshape(-1)
    dirs = _sc_gather_dirs(vert_planar, idx_t)
    return _tc_dense(dirs, directions.T, distance.T)


# speedup vs baseline: 1.0171x; 1.0171x over previous
import functools

import jax
import jax.numpy as jnp
from jax import lax
from jax.experimental import pallas as pl
from jax.experimental.pallas import tpu as pltpu
from jax.experimental.pallas import tpu_sc as plsc

_BS, _V, _NB = 2, 10000, 16
_SK, _K = 256, 128
_VB = 5120


def _sc_gather_dirs(vert_planar, idx_t):
    mesh = plsc.VectorSubcoreMesh(core_axis_name="c", subcore_axis_name="s")

    @functools.partial(
        pl.kernel,
        out_type=jax.ShapeDtypeStruct((_BS, _NB, 3, _V), jnp.float32),
        mesh=mesh,
        scratch_types=[
            [pltpu.VMEM((_V,), jnp.float32) for _ in range(3)],
            pltpu.VMEM((_V,), jnp.int32),
            pltpu.VMEM((3, _V), jnp.float32),
        ],
        compiler_params=pltpu.CompilerParams(needs_layout_passes=False),
    )
    def k(vert_hbm, idx_hbm, out_hbm, tabs, idxs, outs):
        cid = lax.axis_index("c")
        sid = lax.axis_index("s")
        w = sid * 2 + cid
        b = w // _NB
        n = w % _NB
        for c in range(3):
            pltpu.sync_copy(vert_hbm.at[pl.ds((b * 3 + c) * _V, _V)], tabs[c])
        pltpu.sync_copy(idx_hbm.at[pl.ds((b * _NB + n) * _V, _V)], idxs)

        def body(i, carry):
            for u in range(5):
                s = (i * 5 + u) * 16
                iv = idxs[pl.ds(s, 16)]
                for c in range(3):
                    g = plsc.load_gather(tabs[c], [iv])
                    outs[c, pl.ds(s, 16)] = g - tabs[c][pl.ds(s, 16)]
            return carry

        lax.fori_loop(0, _V // 80, body, 0)
        pltpu.sync_copy(outs, out_hbm.at[b, n])

    return k(vert_planar, idx_t)


def _tc_dense(dirs, w_t, dw_t):
    nblk = (_V + _VB - 1) // _VB

    def body(dirs_ref, w_ref, dw_ref, out_ref):
        wv = w_ref[...]
        wn = wv / jnp.maximum(
            jnp.sqrt(jnp.sum(wv * wv, axis=1, keepdims=True)), 1e-12
        )
        acc = None
        dist = None
        for n in range(_NB):
            a = dirs_ref[0, n]
            sq = a[0:1, :] ** 2 + a[1:2, :] ** 2 + a[2:3, :] ** 2
            nrm = jnp.sqrt(sq)
            inv = 1.0 / jnp.maximum(nrm, 1e-12)
            th = jnp.dot(
                wn.astype(jnp.bfloat16),
                (a * inv).astype(jnp.bfloat16),
                preferred_element_type=jnp.float32,
            )
            acc = th if acc is None else jnp.maximum(acc, th)
            dist = nrm if dist is None else jnp.maximum(dist, nrm)
        acc = jnp.maximum(acc, 0.0)
        dv = jnp.maximum(dw_ref[...] * dist, 0.0)
        f = acc + dv
        out_ref[0] = (f[:_K, :] + f[_K:, :]).T

    return pl.pallas_call(
        body,
        grid=(_BS, nblk),
        in_specs=[
            pl.BlockSpec((1, _NB, 3, _VB), lambda b, i: (b, 0, 0, i)),
            pl.BlockSpec((_SK, 3), lambda b, i: (0, 0)),
            pl.BlockSpec((_SK, 1), lambda b, i: (0, 0)),
        ],
        out_specs=pl.BlockSpec((1, _VB, _K), lambda b, i: (b, i, 0)),
        out_shape=jax.ShapeDtypeStruct((_BS, _V, _K), jnp.float32),
    )(dirs, w_t, dw_t)


def kernel(neighbor_index, vertices, directions, distance):
    vert_planar = vertices.transpose(0, 2, 1).reshape(-1)
    idx_t = neighbor_index.transpose(0, 2, 1).astype(jnp.int32).reshape(-1)
    dirs = _sc_gather_dirs(vert_planar, idx_t)
    return _tc_dense(dirs, directions.T, distance.T)
